# baseline (speedup 1.0000x reference)
import dataclasses
import functools

import jax
import jax.numpy as jnp
from jax import lax
from jax.experimental import pallas as pl
from jax.experimental.pallas import tpu as pltpu
from jax.experimental.pallas import tpu_sc as plsc

N = 10000
E = 160000
LANES = 16
NS = 16
NC = 2
EB = 64
EROWS = E // EB
KA = 160
KB = EROWS - KA * (NS - 1)
ZCH = 624
RA = 632
RB = N - RA * (NS - 1)


def _mm_kernel(x_ref, w_ref, b_ref, o_ref):
    acc = jnp.dot(x_ref[...], w_ref[...], preferred_element_type=jnp.float32)
    o_ref[...] = (acc + b_ref[0])[None]


def _mm_relu_acc_kernel(s_ref, w_ref, b_ref, o_ref):
    k = pl.program_id(2)

    @pl.when(k == 0)
    def _():
        o_ref[...] = jnp.broadcast_to(b_ref[...], o_ref.shape)

    h = jnp.maximum(s_ref[0], 0.0)
    o_ref[...] += jnp.dot(h, w_ref[...], preferred_element_type=jnp.float32)[None]


def _small_mm_kernel(a_ref, b_ref, o_ref):
    o_ref[...] = jnp.dot(a_ref[...], b_ref[...], preferred_element_type=jnp.float32)


def _small_mm(a, b):
    return pl.pallas_call(
        _small_mm_kernel,
        out_shape=jax.ShapeDtypeStruct((a.shape[0], b.shape[1]), jnp.float32),
    )(a, b)


def _fused_matmul(x, wcat, biascat, nout):
    kdim = x.shape[1]
    bn = 1000
    return pl.pallas_call(
        _mm_kernel,
        grid=(N // bn, nout),
        in_specs=[
            pl.BlockSpec((bn, kdim), lambda i, j: (i, 0)),
            pl.BlockSpec((kdim, 128), lambda i, j: (0, j)),
            pl.BlockSpec((1, 1, 128), lambda i, j: (j, 0, 0)),
        ],
        out_specs=pl.BlockSpec((1, bn, 128), lambda i, j: (j, i, 0)),
        out_shape=jax.ShapeDtypeStruct((nout, N, 128), jnp.float32),
    )(x, wcat, biascat)


def _fused_matmul_relu(scat, wcat, biascat, nout):
    nk = scat.shape[0]
    bn = 1000
    return pl.pallas_call(
        _mm_relu_acc_kernel,
        grid=(N // bn, nout, nk),
        in_specs=[
            pl.BlockSpec((1, bn, 128), lambda i, j, k: (k, i, 0)),
            pl.BlockSpec((128, 128), lambda i, j, k: (k, j)),
            pl.BlockSpec((1, 1, 128), lambda i, j, k: (j, 0, 0)),
        ],
        out_specs=pl.BlockSpec((1, bn, 128), lambda i, j, k: (j, i, 0)),
        out_shape=jax.ShapeDtypeStruct((nout, N, 128), jnp.float32),
    )(scat, wcat, biascat)


def _gat_edge_sc(hcat, asrc, adst, esrc2d, edst2d, nchunks, colmajor_out):
    cps = nchunks // NC
    if colmajor_out:
        out_type = jax.ShapeDtypeStruct((N, nchunks * 128), jnp.float32)
    else:
        out_type = jax.ShapeDtypeStruct((nchunks, N, 128), jnp.float32)
    mesh = plsc.VectorSubcoreMesh(
        core_axis_name="c", subcore_axis_name="s", num_cores=NC, num_subcores=NS
    )
    cp = pltpu.CompilerParams()
    if "needs_layout_passes" in pltpu.CompilerParams.__dataclass_fields__:
        cp = dataclasses.replace(cp, needs_layout_passes=False)

    @functools.partial(
        pl.kernel,
        out_type=out_type,
        mesh=mesh,
        compiler_params=cp,
        scratch_types=[
            pltpu.VMEM((KA * EB,), jnp.int32),
            pltpu.VMEM((KA * EB,), jnp.int32),
            pltpu.VMEM((KA * EB,), jnp.float32),
            pltpu.VMEM((2, EB), jnp.float32),
            pltpu.VMEM((2, EB), jnp.float32),
            pltpu.VMEM((EB, 128), jnp.float32),
            pltpu.VMEM((EB, 128), jnp.float32),
            pltpu.VMEM((ZCH + LANES,), jnp.float32),
            pltpu.VMEM_SHARED((N, 128), jnp.float32),
            pltpu.VMEM_SHARED((N,), jnp.float32),
            pltpu.SemaphoreType.DMA,
            pltpu.SemaphoreType.DMA,
            pltpu.SemaphoreType.DMA,
            pltpu.SemaphoreType.DMA,
            pltpu.SemaphoreType.DMA,
        ],
    )
    def edge_kernel(hcat_hbm, asrc_hbm, adst_hbm, esrc_hbm, edst_hbm, out_hbm,
                    src1d, dst1d, ex1d, tmpa, tmpb, rows0, rows1, zbuf,
                    acc, denom_sh, gsem0, gsem1, ssem0, ssem1, isem):
        c = lax.axis_index("c")
        s = lax.axis_index("s")
        zero16 = jnp.zeros((LANES,), jnp.float32)
        kk = jnp.where(s < NS - 1, KA, KB)

        @pl.when(s < NS - 1)
        def _():
            pltpu.sync_copy(esrc_hbm.at[pl.ds(s * KA * EB, KA * EB)], src1d)
            pltpu.sync_copy(edst_hbm.at[pl.ds(s * KA * EB, KA * EB)], dst1d)

        @pl.when(s == NS - 1)
        def _():
            pltpu.sync_copy(esrc_hbm.at[pl.ds((NS - 1) * KA * EB, KB * EB)],
                            src1d.at[pl.ds(0, KB * EB)])
            pltpu.sync_copy(edst_hbm.at[pl.ds((NS - 1) * KA * EB, KB * EB)],
                            dst1d.at[pl.ds(0, KB * EB)])

        q0 = c * cps

        @pl.when(s < NS - 1)
        def _():
            pltpu.async_copy(hcat_hbm.at[nchunks + q0, pl.ds(s * RA, RA)],
                             acc.at[pl.ds(s * RA, RA)], isem)

        @pl.when(s == NS - 1)
        def _():
            pltpu.async_copy(hcat_hbm.at[nchunks + q0, pl.ds((NS - 1) * RA, RB)],
                             acc.at[pl.ds((NS - 1) * RA, RB)], isem)

        for zz in range((ZCH + LANES) // LANES):
            zbuf[pl.ds(zz * LANES, LANES)] = zero16

        @pl.when(s < NS - 1)
        def _():
            pltpu.sync_copy(zbuf.at[pl.ds(0, ZCH)],
                            denom_sh.at[pl.ds(s * ZCH, ZCH)])

        @pl.when(s == NS - 1)
        def _():
            pltpu.sync_copy(zbuf.at[pl.ds(0, ZCH + LANES)],
                            denom_sh.at[pl.ds((NS - 1) * ZCH, ZCH + LANES)])

        def p1_issue(j, bank, sem):
            pltpu.async_copy(asrc_hbm.at[src1d.at[pl.ds(j * EB, EB)]],
                             tmpa.at[bank], sem)
            pltpu.async_copy(adst_hbm.at[dst1d.at[pl.ds(j * EB, EB)]],
                             tmpb.at[bank], sem)

        def p1_wait(j, bank, sem):
            pltpu.make_async_copy(asrc_hbm.at[src1d.at[pl.ds(j * EB, EB)]],
                                  tmpa.at[bank], sem).wait()
            pltpu.make_async_copy(adst_hbm.at[dst1d.at[pl.ds(j * EB, EB)]],
                                  tmpb.at[bank], sem).wait()

        def p1_compute(j, bank):
            eb = j * EB
            for cc in range(EB // LANES):
                sl = pl.ds(cc * LANES, LANES)
                e = tmpa[bank, sl] + tmpb[bank, sl]
                e = jnp.maximum(e, 0.2 * e)
                ex1d[pl.ds(eb + cc * LANES, LANES)] = jnp.exp(e)

        kk2 = kk // 2
        p1_issue(0, 0, gsem0)

        @pl.loop(0, kk2)
        def _(p):
            j0 = 2 * p
            j1 = j0 + 1
            p1_issue(j1, 1, gsem1)
            p1_wait(j0, 0, gsem0)
            p1_compute(j0, 0)

            @pl.when(p + 1 < kk2)
            def _():
                p1_issue(j0 + 2, 0, gsem0)

            p1_wait(j1, 1, gsem1)
            p1_compute(j1, 1)

        plsc.subcore_barrier()

        PW = 20

        @pl.loop(0, kk, step=PW)
        def _(j0):
            for dj in range(PW):
                eb = (j0 + dj) * EB
                pltpu.async_copy(ex1d.at[pl.ds(eb, EB)],
                                 denom_sh.at[dst1d.at[pl.ds(eb, EB)]],
                                 ssem0, add=True)
            for dj in range(PW):
                eb = (j0 + dj) * EB
                pltpu.make_async_copy(ex1d.at[pl.ds(eb, EB)],
                                      denom_sh.at[dst1d.at[pl.ds(eb, EB)]],
                                      ssem0).wait()

        @pl.when(s < NS - 1)
        def _():
            pltpu.make_async_copy(hcat_hbm.at[nchunks + q0, pl.ds(s * RA, RA)],
                                  acc.at[pl.ds(s * RA, RA)], isem).wait()

        @pl.when(s == NS - 1)
        def _():
            pltpu.make_async_copy(hcat_hbm.at[nchunks + q0, pl.ds((NS - 1) * RA, RB)],
                                  acc.at[pl.ds((NS - 1) * RA, RB)], isem).wait()

        plsc.subcore_barrier()

        @pl.loop(0, kk)
        def _(j):
            eb = j * EB
            pltpu.sync_copy(denom_sh.at[dst1d.at[pl.ds(eb, EB)]],
                            tmpa.at[0])
            for cc in range(EB // LANES):
                sl = pl.ds(cc * LANES, LANES)
                sle = pl.ds(eb + cc * LANES, LANES)
                ex1d[sle] = ex1d[sle] / (tmpa[0, sl] + 1e-16)

        for i in range(cps):
            q = c * cps + i

            def g_issue(j, buf, sem):
                pltpu.async_copy(hcat_hbm.at[q].at[src1d.at[pl.ds(j * EB, EB)]],
                                 buf, sem)

            def g_wait(j, buf, sem):
                pltpu.make_async_copy(hcat_hbm.at[q].at[src1d.at[pl.ds(j * EB, EB)]],
                                      buf, sem).wait()

            def s_issue(j, buf, sem):
                pltpu.async_copy(buf, acc.at[dst1d.at[pl.ds(j * EB, EB)]],
                                 sem, add=True)

            def s_wait(j, buf, sem):
                pltpu.make_async_copy(buf, acc.at[dst1d.at[pl.ds(j * EB, EB)]],
                                      sem).wait()

            def scale(j, buf):
                eb = j * EB

                @pl.loop(0, EB, unroll=8)
                def _(r):
                    rv = jnp.full((LANES,), eb + r, jnp.int32)
                    a = plsc.load_gather(ex1d, [rv])
                    for cc in range(128 // LANES):
                        sl = pl.ds(cc * LANES, LANES)
                        buf[r, sl] = buf[r, sl] * a

            if i > 0:
                @pl.when(s < NS - 1)
                def _():
                    pltpu.sync_copy(hcat_hbm.at[nchunks + q, pl.ds(s * RA, RA)],
                                    acc.at[pl.ds(s * RA, RA)])

                @pl.when(s == NS - 1)
                def _():
                    pltpu.sync_copy(hcat_hbm.at[nchunks + q, pl.ds((NS - 1) * RA, RB)],
                                    acc.at[pl.ds((NS - 1) * RA, RB)])

                plsc.subcore_barrier()

            kk2 = kk // 2
            g_issue(0, rows0, gsem0)

            @pl.loop(0, kk2)
            def _(p):
                j0 = 2 * p
                j1 = j0 + 1
                g_wait(j0, rows0, gsem0)

                @pl.when(p > 0)
                def _():
                    s_wait(j1 - 2, rows1, ssem1)

                g_issue(j1, rows1, gsem1)
                scale(j0, rows0)
                s_issue(j0, rows0, ssem0)
                g_wait(j1, rows1, gsem1)
                scale(j1, rows1)

                @pl.when(p + 1 < kk2)
                def _():
                    s_wait(j0, rows0, ssem0)
                    g_issue(j0 + 2, rows0, gsem0)

                s_issue(j1, rows1, ssem1)

            s_wait(kk - 2, rows0, ssem0)
            s_wait(kk - 1, rows1, ssem1)

            plsc.subcore_barrier()
            if colmajor_out:
                @pl.when(s < NS - 1)
                def _():
                    pltpu.sync_copy(
                        acc.at[pl.ds(s * RA, RA)],
                        out_hbm.at[pl.ds(s * RA, RA), pl.ds(q * 128, 128)])

                @pl.when(s == NS - 1)
                def _():
                    pltpu.sync_copy(
                        acc.at[pl.ds((NS - 1) * RA, RB)],
                        out_hbm.at[pl.ds((NS - 1) * RA, RB), pl.ds(q * 128, 128)])
            else:
                @pl.when(s < NS - 1)
                def _():
                    pltpu.sync_copy(acc.at[pl.ds(s * RA, RA)],
                                    out_hbm.at[q, pl.ds(s * RA, RA)])

                @pl.when(s == NS - 1)
                def _():
                    pltpu.sync_copy(acc.at[pl.ds((NS - 1) * RA, RB)],
                                    out_hbm.at[q, pl.ds((NS - 1) * RA, RB)])
            plsc.subcore_barrier()

    return edge_kernel(hcat, asrc, adst, esrc2d, edst2d)


def kernel(x, edge_index, W1, a_src1, a_dst1, b1, Wl1, bl1,
           W2, a_src2, a_dst2, b2, Wl2, bl2):
    esrc2d = edge_index[0].astype(jnp.int32)
    edst2d = edge_index[1].astype(jnp.int32)

    a1 = jnp.zeros((512, 128), jnp.float32).at[:, 0].set(a_src1).at[:, 1].set(a_dst1)
    av1 = _small_mm(W1, a1)
    wcat1 = jnp.concatenate([W1, Wl1, av1], axis=1)
    bias1 = jnp.zeros((9, 1, 128), jnp.float32).at[4:8, 0, :].set(
        (b1 + bl1).reshape(4, 128))
    h1cat = _fused_matmul(x, wcat1, bias1, 9)
    asrc1 = h1cat[8, :, 0]
    adst1 = h1cat[8, :, 1]

    s1 = _gat_edge_sc(h1cat, asrc1, adst1, esrc2d, edst2d, 4, colmajor_out=False)

    a2 = jnp.zeros((256, 128), jnp.float32).at[:, 0].set(a_src2).at[:, 1].set(a_dst2)
    av2 = _small_mm(W2, a2)
    wcat2 = jnp.concatenate([W2, Wl2, av2], axis=1)
    bias2 = jnp.zeros((5, 1, 128), jnp.float32).at[2:4, 0, :].set(
        (b2 + bl2).reshape(2, 128))
    h2cat = _fused_matmul_relu(s1, wcat2, bias2, 5)
    asrc2 = h2cat[4, :, 0]
    adst2 = h2cat[4, :, 1]

    out = _gat_edge_sc(h2cat, asrc2, adst2, esrc2d, edst2d, 2, colmajor_out=True)
    return out


# speedup vs baseline: 1.0028x; 1.0028x over previous
import dataclasses
import functools

import jax
import jax.numpy as jnp
from jax import lax
from jax.experimental import pallas as pl
from jax.experimental.pallas import tpu as pltpu
from jax.experimental.pallas import tpu_sc as plsc

N = 10000
E = 160000
LANES = 16
NS = 16
NC = 2
EB = 64
EROWS = E // EB
KA = 160
KB = EROWS - KA * (NS - 1)
ZCH = 624
RA = 632
RB = N - RA * (NS - 1)


def _mm_kernel(x_ref, w_ref, b_ref, o_ref):
    acc = jnp.dot(x_ref[...], w_ref[...], preferred_element_type=jnp.float32)
    o_ref[...] = (acc + b_ref[0])[None]


def _mm_relu_acc_kernel(s_ref, w_ref, b_ref, o_ref):
    k = pl.program_id(2)

    @pl.when(k == 0)
    def _():
        o_ref[...] = jnp.broadcast_to(b_ref[...], o_ref.shape)

    h = jnp.maximum(s_ref[0], 0.0)
    o_ref[...] += jnp.dot(h, w_ref[...], preferred_element_type=jnp.float32)[None]


def _small_mm_kernel(a_ref, b_ref, o_ref):
    o_ref[...] = jnp.dot(a_ref[...], b_ref[...], preferred_element_type=jnp.float32)


def _small_mm(a, b):
    return pl.pallas_call(
        _small_mm_kernel,
        out_shape=jax.ShapeDtypeStruct((a.shape[0], b.shape[1]), jnp.float32),
    )(a, b)


def _fused_matmul(x, wcat, biascat, nout):
    kdim = x.shape[1]
    bn = 1000
    return pl.pallas_call(
        _mm_kernel,
        grid=(N // bn, nout),
        in_specs=[
            pl.BlockSpec((bn, kdim), lambda i, j: (i, 0)),
            pl.BlockSpec((kdim, 128), lambda i, j: (0, j)),
            pl.BlockSpec((1, 1, 128), lambda i, j: (j, 0, 0)),
        ],
        out_specs=pl.BlockSpec((1, bn, 128), lambda i, j: (j, i, 0)),
        out_shape=jax.ShapeDtypeStruct((nout, N, 128), jnp.float32),
    )(x, wcat, biascat)


def _fused_matmul_relu(scat, wcat, biascat, nout):
    nk = scat.shape[0]
    bn = 1000
    return pl.pallas_call(
        _mm_relu_acc_kernel,
        grid=(N // bn, nout, nk),
        in_specs=[
            pl.BlockSpec((1, bn, 128), lambda i, j, k: (k, i, 0)),
            pl.BlockSpec((128, 128), lambda i, j, k: (k, j)),
            pl.BlockSpec((1, 1, 128), lambda i, j, k: (j, 0, 0)),
        ],
        out_specs=pl.BlockSpec((1, bn, 128), lambda i, j, k: (j, i, 0)),
        out_shape=jax.ShapeDtypeStruct((nout, N, 128), jnp.float32),
    )(scat, wcat, biascat)


def _gat_edge_sc(hcat, asrc, adst, esrc2d, edst2d, nchunks, colmajor_out):
    cps = nchunks // NC
    if colmajor_out:
        out_type = jax.ShapeDtypeStruct((N, nchunks * 128), jnp.float32)
    else:
        out_type = jax.ShapeDtypeStruct((nchunks, N, 128), jnp.float32)
    mesh = plsc.VectorSubcoreMesh(
        core_axis_name="c", subcore_axis_name="s", num_cores=NC, num_subcores=NS
    )
    cp = pltpu.CompilerParams()
    if "needs_layout_passes" in pltpu.CompilerParams.__dataclass_fields__:
        cp = dataclasses.replace(cp, needs_layout_passes=False)

    @functools.partial(
        pl.kernel,
        out_type=out_type,
        mesh=mesh,
        compiler_params=cp,
        scratch_types=[
            pltpu.VMEM((KA * EB,), jnp.int32),
            pltpu.VMEM((KA * EB,), jnp.int32),
            pltpu.VMEM((KA * EB,), jnp.float32),
            pltpu.VMEM((2, EB), jnp.float32),
            pltpu.VMEM((2, EB), jnp.float32),
            pltpu.VMEM((EB, 128), jnp.float32),
            pltpu.VMEM((EB, 128), jnp.float32),
            pltpu.VMEM((ZCH + LANES,), jnp.float32),
            pltpu.VMEM_SHARED((N, 128), jnp.float32),
            pltpu.VMEM_SHARED((N,), jnp.float32),
            pltpu.SemaphoreType.DMA,
            pltpu.SemaphoreType.DMA,
            pltpu.SemaphoreType.DMA,
            pltpu.SemaphoreType.DMA,
            pltpu.SemaphoreType.DMA,
        ],
    )
    def edge_kernel(hcat_hbm, asrc_hbm, adst_hbm, esrc_hbm, edst_hbm, out_hbm,
                    src1d, dst1d, ex1d, tmpa, tmpb, rows0, rows1, zbuf,
                    acc, denom_sh, gsem0, gsem1, ssem0, ssem1, isem):
        c = lax.axis_index("c")
        s = lax.axis_index("s")
        zero16 = jnp.zeros((LANES,), jnp.float32)
        kk = jnp.where(s < NS - 1, KA, KB)

        @pl.when(s < NS - 1)
        def _():
            pltpu.sync_copy(esrc_hbm.at[pl.ds(s * KA * EB, KA * EB)], src1d)
            pltpu.sync_copy(edst_hbm.at[pl.ds(s * KA * EB, KA * EB)], dst1d)

        @pl.when(s == NS - 1)
        def _():
            pltpu.sync_copy(esrc_hbm.at[pl.ds((NS - 1) * KA * EB, KB * EB)],
                            src1d.at[pl.ds(0, KB * EB)])
            pltpu.sync_copy(edst_hbm.at[pl.ds((NS - 1) * KA * EB, KB * EB)],
                            dst1d.at[pl.ds(0, KB * EB)])

        q0 = c * cps

        @pl.when(s < NS - 1)
        def _():
            pltpu.async_copy(hcat_hbm.at[nchunks + q0, pl.ds(s * RA, RA)],
                             acc.at[pl.ds(s * RA, RA)], isem)

        @pl.when(s == NS - 1)
        def _():
            pltpu.async_copy(hcat_hbm.at[nchunks + q0, pl.ds((NS - 1) * RA, RB)],
                             acc.at[pl.ds((NS - 1) * RA, RB)], isem)

        for zz in range((ZCH + LANES) // LANES):
            zbuf[pl.ds(zz * LANES, LANES)] = zero16

        @pl.when(s < NS - 1)
        def _():
            pltpu.sync_copy(zbuf.at[pl.ds(0, ZCH)],
                            denom_sh.at[pl.ds(s * ZCH, ZCH)])

        @pl.when(s == NS - 1)
        def _():
            pltpu.sync_copy(zbuf.at[pl.ds(0, ZCH + LANES)],
                            denom_sh.at[pl.ds((NS - 1) * ZCH, ZCH + LANES)])

        def p1_issue(j, bank, sem):
            pltpu.async_copy(asrc_hbm.at[src1d.at[pl.ds(j * EB, EB)]],
                             tmpa.at[bank], sem)
            pltpu.async_copy(adst_hbm.at[dst1d.at[pl.ds(j * EB, EB)]],
                             tmpb.at[bank], sem)

        def p1_wait(j, bank, sem):
            pltpu.make_async_copy(asrc_hbm.at[src1d.at[pl.ds(j * EB, EB)]],
                                  tmpa.at[bank], sem).wait()
            pltpu.make_async_copy(adst_hbm.at[dst1d.at[pl.ds(j * EB, EB)]],
                                  tmpb.at[bank], sem).wait()

        def p1_compute(j, bank):
            eb = j * EB
            for cc in range(EB // LANES):
                sl = pl.ds(cc * LANES, LANES)
                e = tmpa[bank, sl] + tmpb[bank, sl]
                e = jnp.maximum(e, 0.2 * e)
                ex1d[pl.ds(eb + cc * LANES, LANES)] = jnp.exp(e)

        kk2 = kk // 2
        p1_issue(0, 0, gsem0)

        @pl.loop(0, kk2)
        def _(p):
            j0 = 2 * p
            j1 = j0 + 1
            p1_issue(j1, 1, gsem1)
            p1_wait(j0, 0, gsem0)
            p1_compute(j0, 0)

            @pl.when(p + 1 < kk2)
            def _():
                p1_issue(j0 + 2, 0, gsem0)

            p1_wait(j1, 1, gsem1)
            p1_compute(j1, 1)

        plsc.subcore_barrier()

        PW = 20

        @pl.loop(0, kk, step=PW)
        def _(j0):
            for dj in range(PW):
                eb = (j0 + dj) * EB
                pltpu.async_copy(ex1d.at[pl.ds(eb, EB)],
                                 denom_sh.at[dst1d.at[pl.ds(eb, EB)]],
                                 ssem0, add=True)
            for dj in range(PW):
                eb = (j0 + dj) * EB
                pltpu.make_async_copy(ex1d.at[pl.ds(eb, EB)],
                                      denom_sh.at[dst1d.at[pl.ds(eb, EB)]],
                                      ssem0).wait()

        @pl.when(s < NS - 1)
        def _():
            pltpu.make_async_copy(hcat_hbm.at[nchunks + q0, pl.ds(s * RA, RA)],
                                  acc.at[pl.ds(s * RA, RA)], isem).wait()

        @pl.when(s == NS - 1)
        def _():
            pltpu.make_async_copy(hcat_hbm.at[nchunks + q0, pl.ds((NS - 1) * RA, RB)],
                                  acc.at[pl.ds((NS - 1) * RA, RB)], isem).wait()

        plsc.subcore_barrier()

        @pl.loop(0, kk)
        def _(j):
            eb = j * EB
            pltpu.sync_copy(denom_sh.at[dst1d.at[pl.ds(eb, EB)]],
                            tmpa.at[0])
            for cc in range(EB // LANES):
                sl = pl.ds(cc * LANES, LANES)
                sle = pl.ds(eb + cc * LANES, LANES)
                ex1d[sle] = ex1d[sle] / (tmpa[0, sl] + 1e-16)

        for i in range(cps):
            q = c * cps + i

            def g_issue(j, buf, sem):
                pltpu.async_copy(hcat_hbm.at[q].at[src1d.at[pl.ds(j * EB, EB)]],
                                 buf, sem)

            def g_wait(j, buf, sem):
                pltpu.make_async_copy(hcat_hbm.at[q].at[src1d.at[pl.ds(j * EB, EB)]],
                                      buf, sem).wait()

            def s_issue(j, buf, sem):
                pltpu.async_copy(buf, acc.at[dst1d.at[pl.ds(j * EB, EB)]],
                                 sem, add=True)

            def s_wait(j, buf, sem):
                pltpu.make_async_copy(buf, acc.at[dst1d.at[pl.ds(j * EB, EB)]],
                                      sem).wait()

            def scale(j, buf):
                eb = j * EB

                @pl.loop(0, EB, unroll=4)
                def _(r):
                    rv = jnp.full((LANES,), eb + r, jnp.int32)
                    a = plsc.load_gather(ex1d, [rv])
                    for cc in range(128 // LANES):
                        sl = pl.ds(cc * LANES, LANES)
                        buf[r, sl] = buf[r, sl] * a

            if i > 0:
                @pl.when(s < NS - 1)
                def _():
                    pltpu.sync_copy(hcat_hbm.at[nchunks + q, pl.ds(s * RA, RA)],
                                    acc.at[pl.ds(s * RA, RA)])

                @pl.when(s == NS - 1)
                def _():
                    pltpu.sync_copy(hcat_hbm.at[nchunks + q, pl.ds((NS - 1) * RA, RB)],
                                    acc.at[pl.ds((NS - 1) * RA, RB)])

                plsc.subcore_barrier()

            kk2 = kk // 2
            g_issue(0, rows0, gsem0)

            @pl.loop(0, kk2)
            def _(p):
                j0 = 2 * p
                j1 = j0 + 1
                g_wait(j0, rows0, gsem0)

                @pl.when(p > 0)
                def _():
                    s_wait(j1 - 2, rows1, ssem1)

                g_issue(j1, rows1, gsem1)
                scale(j0, rows0)
                s_issue(j0, rows0, ssem0)
                g_wait(j1, rows1, gsem1)
                scale(j1, rows1)

                @pl.when(p + 1 < kk2)
                def _():
                    s_wait(j0, rows0, ssem0)
                    g_issue(j0 + 2, rows0, gsem0)

                s_issue(j1, rows1, ssem1)

            s_wait(kk - 2, rows0, ssem0)
            s_wait(kk - 1, rows1, ssem1)

            plsc.subcore_barrier()
            if colmajor_out:
                @pl.when(s < NS - 1)
                def _():
                    pltpu.sync_copy(
                        acc.at[pl.ds(s * RA, RA)],
                        out_hbm.at[pl.ds(s * RA, RA), pl.ds(q * 128, 128)])

                @pl.when(s == NS - 1)
                def _():
                    pltpu.sync_copy(
                        acc.at[pl.ds((NS - 1) * RA, RB)],
                        out_hbm.at[pl.ds((NS - 1) * RA, RB), pl.ds(q * 128, 128)])
            else:
                @pl.when(s < NS - 1)
                def _():
                    pltpu.sync_copy(acc.at[pl.ds(s * RA, RA)],
                                    out_hbm.at[q, pl.ds(s * RA, RA)])

                @pl.when(s == NS - 1)
                def _():
                    pltpu.sync_copy(acc.at[pl.ds((NS - 1) * RA, RB)],
                                    out_hbm.at[q, pl.ds((NS - 1) * RA, RB)])
            plsc.subcore_barrier()

    return edge_kernel(hcat, asrc, adst, esrc2d, edst2d)


def kernel(x, edge_index, W1, a_src1, a_dst1, b1, Wl1, bl1,
           W2, a_src2, a_dst2, b2, Wl2, bl2):
    esrc2d = edge_index[0].astype(jnp.int32)
    edst2d = edge_index[1].astype(jnp.int32)

    a1 = jnp.zeros((512, 128), jnp.float32).at[:, 0].set(a_src1).at[:, 1].set(a_dst1)
    av1 = _small_mm(W1, a1)
    wcat1 = jnp.concatenate([W1, Wl1, av1], axis=1)
    bias1 = jnp.zeros((9, 1, 128), jnp.float32).at[4:8, 0, :].set(
        (b1 + bl1).reshape(4, 128))
    h1cat = _fused_matmul(x, wcat1, bias1, 9)
    asrc1 = h1cat[8, :, 0]
    adst1 = h1cat[8, :, 1]

    s1 = _gat_edge_sc(h1cat, asrc1, adst1, esrc2d, edst2d, 4, colmajor_out=False)

    a2 = jnp.zeros((256, 128), jnp.float32).at[:, 0].set(a_src2).at[:, 1].set(a_dst2)
    av2 = _small_mm(W2, a2)
    wcat2 = jnp.concatenate([W2, Wl2, av2], axis=1)
    bias2 = jnp.zeros((5, 1, 128), jnp.float32).at[2:4, 0, :].set(
        (b2 + bl2).reshape(2, 128))
    h2cat = _fused_matmul_relu(s1, wcat2, bias2, 5)
    asrc2 = h2cat[4, :, 0]
    adst2 = h2cat[4, :, 1]

    out = _gat_edge_sc(h2cat, asrc2, adst2, esrc2d, edst2d, 2, colmajor_out=True)
    return out


# speedup vs baseline: 1.0601x; 1.0572x over previous
import dataclasses
import functools

import jax
import jax.numpy as jnp
from jax import lax
from jax.experimental import pallas as pl
from jax.experimental.pallas import tpu as pltpu
from jax.experimental.pallas import tpu_sc as plsc

N = 10000
E = 160000
LANES = 16
NS = 16
NC = 2
EB = 64
EROWS = E // EB
KA = 160
KB = EROWS - KA * (NS - 1)
ZCH = 624
RA = 632
RB = N - RA * (NS - 1)


def _mm_kernel(x_ref, w_ref, b_ref, o_ref):
    acc = jnp.dot(x_ref[...], w_ref[...], preferred_element_type=jnp.float32)
    o_ref[...] = (acc + b_ref[0])[None]


def _mm_relu_acc_kernel(s_ref, w_ref, b_ref, o_ref):
    k = pl.program_id(2)

    @pl.when(k == 0)
    def _():
        o_ref[...] = jnp.broadcast_to(b_ref[...], o_ref.shape)

    h = jnp.maximum(s_ref[0], 0.0)
    o_ref[...] += jnp.dot(h, w_ref[...], preferred_element_type=jnp.float32)[None]


def _small_mm_kernel(a_ref, b_ref, o_ref):
    o_ref[...] = jnp.dot(a_ref[...], b_ref[...], preferred_element_type=jnp.float32)


def _small_mm(a, b):
    return pl.pallas_call(
        _small_mm_kernel,
        out_shape=jax.ShapeDtypeStruct((a.shape[0], b.shape[1]), jnp.float32),
    )(a, b)


def _fused_matmul(x, wcat, biascat, nout):
    kdim = x.shape[1]
    bn = 1000
    return pl.pallas_call(
        _mm_kernel,
        grid=(N // bn, nout),
        in_specs=[
            pl.BlockSpec((bn, kdim), lambda i, j: (i, 0)),
            pl.BlockSpec((kdim, 128), lambda i, j: (0, j)),
            pl.BlockSpec((1, 1, 128), lambda i, j: (j, 0, 0)),
        ],
        out_specs=pl.BlockSpec((1, bn, 128), lambda i, j: (j, i, 0)),
        out_shape=jax.ShapeDtypeStruct((nout, N, 128), jnp.float32),
    )(x, wcat, biascat)


def _fused_matmul_relu(scat, wcat, biascat, nout):
    nk = scat.shape[0]
    bn = 1000
    return pl.pallas_call(
        _mm_relu_acc_kernel,
        grid=(N // bn, nout, nk),
        in_specs=[
            pl.BlockSpec((1, bn, 128), lambda i, j, k: (k, i, 0)),
            pl.BlockSpec((128, 128), lambda i, j, k: (k, j)),
            pl.BlockSpec((1, 1, 128), lambda i, j, k: (j, 0, 0)),
        ],
        out_specs=pl.BlockSpec((1, bn, 128), lambda i, j, k: (j, i, 0)),
        out_shape=jax.ShapeDtypeStruct((nout, N, 128), jnp.float32),
    )(scat, wcat, biascat)


def _gat_edge_sc(hcat, asrc, adst, esrc2d, edst2d, nchunks, colmajor_out):
    cps = nchunks // NC
    if colmajor_out:
        out_type = jax.ShapeDtypeStruct((N, nchunks * 128), jnp.float32)
    else:
        out_type = jax.ShapeDtypeStruct((nchunks, N, 128), jnp.float32)
    mesh = plsc.VectorSubcoreMesh(
        core_axis_name="c", subcore_axis_name="s", num_cores=NC, num_subcores=NS
    )
    cp = pltpu.CompilerParams()
    if "needs_layout_passes" in pltpu.CompilerParams.__dataclass_fields__:
        cp = dataclasses.replace(cp, needs_layout_passes=False)

    @functools.partial(
        pl.kernel,
        out_type=out_type,
        mesh=mesh,
        compiler_params=cp,
        scratch_types=[
            pltpu.VMEM((KA * EB,), jnp.int32),
            pltpu.VMEM((KA * EB,), jnp.int32),
            pltpu.VMEM((KA * EB,), jnp.float32),
            pltpu.VMEM((2, EB), jnp.float32),
            pltpu.VMEM((2, EB), jnp.float32),
            pltpu.VMEM((EB, 128), jnp.float32),
            pltpu.VMEM((EB, 128), jnp.float32),
            pltpu.VMEM((ZCH + LANES,), jnp.float32),
            pltpu.VMEM_SHARED((N, 128), jnp.float32),
            pltpu.VMEM_SHARED((N,), jnp.float32),
            pltpu.SemaphoreType.DMA,
            pltpu.SemaphoreType.DMA,
            pltpu.SemaphoreType.DMA,
            pltpu.SemaphoreType.DMA,
            pltpu.SemaphoreType.DMA,
        ],
    )
    def edge_kernel(hcat_hbm, asrc_hbm, adst_hbm, esrc_hbm, edst_hbm, out_hbm,
                    src1d, dst1d, ex1d, tmpa, tmpb, rows0, rows1, zbuf,
                    acc, denom_sh, gsem0, gsem1, ssem0, ssem1, isem):
        c = lax.axis_index("c")
        s = lax.axis_index("s")
        zero16 = jnp.zeros((LANES,), jnp.float32)
        kk = jnp.where(s < NS - 1, KA, KB)

        @pl.when(s < NS - 1)
        def _():
            pltpu.sync_copy(esrc_hbm.at[pl.ds(s * KA * EB, KA * EB)], src1d)
            pltpu.sync_copy(edst_hbm.at[pl.ds(s * KA * EB, KA * EB)], dst1d)

        @pl.when(s == NS - 1)
        def _():
            pltpu.sync_copy(esrc_hbm.at[pl.ds((NS - 1) * KA * EB, KB * EB)],
                            src1d.at[pl.ds(0, KB * EB)])
            pltpu.sync_copy(edst_hbm.at[pl.ds((NS - 1) * KA * EB, KB * EB)],
                            dst1d.at[pl.ds(0, KB * EB)])

        q0 = c * cps

        @pl.when(s < NS - 1)
        def _():
            pltpu.async_copy(hcat_hbm.at[nchunks + q0, pl.ds(s * RA, RA)],
                             acc.at[pl.ds(s * RA, RA)], isem)

        @pl.when(s == NS - 1)
        def _():
            pltpu.async_copy(hcat_hbm.at[nchunks + q0, pl.ds((NS - 1) * RA, RB)],
                             acc.at[pl.ds((NS - 1) * RA, RB)], isem)

        for zz in range((ZCH + LANES) // LANES):
            zbuf[pl.ds(zz * LANES, LANES)] = zero16

        @pl.when(s < NS - 1)
        def _():
            pltpu.sync_copy(zbuf.at[pl.ds(0, ZCH)],
                            denom_sh.at[pl.ds(s * ZCH, ZCH)])

        @pl.when(s == NS - 1)
        def _():
            pltpu.sync_copy(zbuf.at[pl.ds(0, ZCH + LANES)],
                            denom_sh.at[pl.ds((NS - 1) * ZCH, ZCH + LANES)])

        def p1_issue(j, bank, sem):
            pltpu.async_copy(asrc_hbm.at[src1d.at[pl.ds(j * EB, EB)]],
                             tmpa.at[bank], sem)
            pltpu.async_copy(adst_hbm.at[dst1d.at[pl.ds(j * EB, EB)]],
                             tmpb.at[bank], sem)

        def p1_wait(j, bank, sem):
            pltpu.make_async_copy(asrc_hbm.at[src1d.at[pl.ds(j * EB, EB)]],
                                  tmpa.at[bank], sem).wait()
            pltpu.make_async_copy(adst_hbm.at[dst1d.at[pl.ds(j * EB, EB)]],
                                  tmpb.at[bank], sem).wait()

        def p1_compute(j, bank):
            eb = j * EB
            for cc in range(EB // LANES):
                sl = pl.ds(cc * LANES, LANES)
                e = tmpa[bank, sl] + tmpb[bank, sl]
                e = jnp.maximum(e, 0.2 * e)
                ex1d[pl.ds(eb + cc * LANES, LANES)] = jnp.exp(e)

        kk2 = kk // 2
        p1_issue(0, 0, gsem0)

        @pl.loop(0, kk2)
        def _(p):
            j0 = 2 * p
            j1 = j0 + 1
            p1_issue(j1, 1, gsem1)
            p1_wait(j0, 0, gsem0)
            p1_compute(j0, 0)

            @pl.when(p + 1 < kk2)
            def _():
                p1_issue(j0 + 2, 0, gsem0)

            p1_wait(j1, 1, gsem1)
            p1_compute(j1, 1)

        plsc.subcore_barrier()

        PW = 20

        @pl.loop(0, kk, step=PW)
        def _(j0):
            for dj in range(PW):
                eb = (j0 + dj) * EB
                pltpu.async_copy(ex1d.at[pl.ds(eb, EB)],
                                 denom_sh.at[dst1d.at[pl.ds(eb, EB)]],
                                 ssem0, add=True)
            for dj in range(PW):
                eb = (j0 + dj) * EB
                pltpu.make_async_copy(ex1d.at[pl.ds(eb, EB)],
                                      denom_sh.at[dst1d.at[pl.ds(eb, EB)]],
                                      ssem0).wait()

        @pl.when(s < NS - 1)
        def _():
            pltpu.make_async_copy(hcat_hbm.at[nchunks + q0, pl.ds(s * RA, RA)],
                                  acc.at[pl.ds(s * RA, RA)], isem).wait()

        @pl.when(s == NS - 1)
        def _():
            pltpu.make_async_copy(hcat_hbm.at[nchunks + q0, pl.ds((NS - 1) * RA, RB)],
                                  acc.at[pl.ds((NS - 1) * RA, RB)], isem).wait()

        plsc.subcore_barrier()

        @pl.loop(0, kk)
        def _(j):
            eb = j * EB
            pltpu.sync_copy(denom_sh.at[dst1d.at[pl.ds(eb, EB)]],
                            tmpa.at[0])
            for cc in range(EB // LANES):
                sl = pl.ds(cc * LANES, LANES)
                sle = pl.ds(eb + cc * LANES, LANES)
                ex1d[sle] = ex1d[sle] / (tmpa[0, sl] + 1e-16)

        for i in range(cps):
            q = c * cps + i

            def g_issue(j, buf, sem):
                pltpu.async_copy(hcat_hbm.at[q].at[src1d.at[pl.ds(j * EB, EB)]],
                                 buf, sem)

            def g_wait(j, buf, sem):
                pltpu.make_async_copy(hcat_hbm.at[q].at[src1d.at[pl.ds(j * EB, EB)]],
                                      buf, sem).wait()

            def s_issue(j, buf, sem):
                pltpu.async_copy(buf, acc.at[dst1d.at[pl.ds(j * EB, EB)]],
                                 sem, add=True)

            def s_wait(j, buf, sem):
                pltpu.make_async_copy(buf, acc.at[dst1d.at[pl.ds(j * EB, EB)]],
                                      sem).wait()

            def scale(j, buf):
                eb = j * EB

                @pl.loop(0, EB // LANES)
                def _(rg):
                    av16 = ex1d[pl.ds(eb + rg * LANES, LANES)]
                    rbase = rg * LANES
                    for rr in range(LANES):
                        a = av16.at[jnp.full((LANES,), rr, jnp.int32)].get(
                            mode="promise_in_bounds")
                        r = rbase + rr
                        for cc in range(128 // LANES):
                            sl = pl.ds(cc * LANES, LANES)
                            buf[r, sl] = buf[r, sl] * a

            if i > 0:
                @pl.when(s < NS - 1)
                def _():
                    pltpu.sync_copy(hcat_hbm.at[nchunks + q, pl.ds(s * RA, RA)],
                                    acc.at[pl.ds(s * RA, RA)])

                @pl.when(s == NS - 1)
                def _():
                    pltpu.sync_copy(hcat_hbm.at[nchunks + q, pl.ds((NS - 1) * RA, RB)],
                                    acc.at[pl.ds((NS - 1) * RA, RB)])

                plsc.subcore_barrier()

            kk2 = kk // 2
            g_issue(0, rows0, gsem0)

            @pl.loop(0, kk2)
            def _(p):
                j0 = 2 * p
                j1 = j0 + 1
                g_wait(j0, rows0, gsem0)

                @pl.when(p > 0)
                def _():
                    s_wait(j1 - 2, rows1, ssem1)

                g_issue(j1, rows1, gsem1)
                scale(j0, rows0)
                s_issue(j0, rows0, ssem0)
                g_wait(j1, rows1, gsem1)
                scale(j1, rows1)

                @pl.when(p + 1 < kk2)
                def _():
                    s_wait(j0, rows0, ssem0)
                    g_issue(j0 + 2, rows0, gsem0)

                s_issue(j1, rows1, ssem1)

            s_wait(kk - 2, rows0, ssem0)
            s_wait(kk - 1, rows1, ssem1)

            plsc.subcore_barrier()
            if colmajor_out:
                @pl.when(s < NS - 1)
                def _():
                    pltpu.sync_copy(
                        acc.at[pl.ds(s * RA, RA)],
                        out_hbm.at[pl.ds(s * RA, RA), pl.ds(q * 128, 128)])

                @pl.when(s == NS - 1)
                def _():
                    pltpu.sync_copy(
                        acc.at[pl.ds((NS - 1) * RA, RB)],
                        out_hbm.at[pl.ds((NS - 1) * RA, RB), pl.ds(q * 128, 128)])
            else:
                @pl.when(s < NS - 1)
                def _():
                    pltpu.sync_copy(acc.at[pl.ds(s * RA, RA)],
                                    out_hbm.at[q, pl.ds(s * RA, RA)])

                @pl.when(s == NS - 1)
                def _():
                    pltpu.sync_copy(acc.at[pl.ds((NS - 1) * RA, RB)],
                                    out_hbm.at[q, pl.ds((NS - 1) * RA, RB)])
            plsc.subcore_barrier()

    return edge_kernel(hcat, asrc, adst, esrc2d, edst2d)


def kernel(x, edge_index, W1, a_src1, a_dst1, b1, Wl1, bl1,
           W2, a_src2, a_dst2, b2, Wl2, bl2):
    esrc2d = edge_index[0].astype(jnp.int32)
    edst2d = edge_index[1].astype(jnp.int32)

    a1 = jnp.zeros((512, 128), jnp.float32).at[:, 0].set(a_src1).at[:, 1].set(a_dst1)
    av1 = _small_mm(W1, a1)
    wcat1 = jnp.concatenate([W1, Wl1, av1], axis=1)
    bias1 = jnp.zeros((9, 1, 128), jnp.float32).at[4:8, 0, :].set(
        (b1 + bl1).reshape(4, 128))
    h1cat = _fused_matmul(x, wcat1, bias1, 9)
    asrc1 = h1cat[8, :, 0]
    adst1 = h1cat[8, :, 1]

    s1 = _gat_edge_sc(h1cat, asrc1, adst1, esrc2d, edst2d, 4, colmajor_out=False)

    a2 = jnp.zeros((256, 128), jnp.float32).at[:, 0].set(a_src2).at[:, 1].set(a_dst2)
    av2 = _small_mm(W2, a2)
    wcat2 = jnp.concatenate([W2, Wl2, av2], axis=1)
    bias2 = jnp.zeros((5, 1, 128), jnp.float32).at[2:4, 0, :].set(
        (b2 + bl2).reshape(2, 128))
    h2cat = _fused_matmul_relu(s1, wcat2, bias2, 5)
    asrc2 = h2cat[4, :, 0]
    adst2 = h2cat[4, :, 1]

    out = _gat_edge_sc(h2cat, asrc2, adst2, esrc2d, edst2d, 2, colmajor_out=True)
    return out
